# baseline (speedup 1.0000x reference)
import jax
import jax.numpy as jnp
from jax.experimental import pallas as pl
from jax.experimental.pallas import tpu as pltpu

N = 10000
D = 128
H = 128
G = 64
SCORE = 10

BN = 2000
NB = N // BN


def _pfn_kernel(x_ref, batch_ref, pw1_ref, pb1_ref, pw2_ref, pb2_ref,
                fw1_ref, fb1_ref, fw2_ref, fb2_ref, out_ref,
                p_ref, cnt_ref):
    i = pl.program_id(0)

    @pl.when(i == 0)
    def _init():
        p_ref[...] = jnp.zeros_like(p_ref)
        cnt_ref[...] = jnp.zeros_like(cnt_ref)

    h1 = jax.lax.dot_general(x_ref[...], pw1_ref[...], (((1,), (0,)), ((), ())),
                             preferred_element_type=jnp.float32)
    h1 = jnp.maximum(h1 + pb1_ref[...], 0.0)
    onehot_t = (batch_ref[0] ==
                jax.lax.broadcasted_iota(jnp.int32, (G, 1), 0)).astype(jnp.bfloat16)
    p_ref[...] += jax.lax.dot_general(onehot_t, h1.astype(jnp.bfloat16),
                                      (((1,), (0,)), ((), ())),
                                      preferred_element_type=jnp.float32)
    cnt_ref[...] += jnp.sum(onehot_t.astype(jnp.float32), axis=1, keepdims=True)

    @pl.when(i == NB - 1)
    def _tail():
        hp = jax.lax.Precision.HIGHEST
        pooled = jax.lax.dot_general(p_ref[...], pw2_ref[...],
                                     (((1,), (0,)), ((), ())),
                                     preferred_element_type=jnp.float32,
                                     precision=hp)
        pooled = pooled + cnt_ref[...] * pb2_ref[...]
        z = jax.lax.dot_general(pooled, fw1_ref[...], (((1,), (0,)), ((), ())),
                                preferred_element_type=jnp.float32,
                                precision=hp)
        z = jnp.maximum(z + fb1_ref[...], 0.0)
        out_ref[...] = jax.lax.dot_general(z, fw2_ref[...],
                                           (((1,), (0,)), ((), ())),
                                           preferred_element_type=jnp.float32,
                                           precision=hp) + fb2_ref[...]


def _full(shape):
    return pl.BlockSpec(shape, lambda i: (0, 0))


@jax.jit
def _run(x, batch2d, phi_W1, phi_b1, phi_W2, phi_b2, f_W1, f_b1, f_W2, f_b2):
    return pl.pallas_call(
        _pfn_kernel,
        grid=(NB,),
        in_specs=[
            pl.BlockSpec((BN, D), lambda i: (i, 0)),
            pl.BlockSpec((1, 1, BN), lambda i: (i, 0, 0)),
            _full((D, H)), _full((1, H)),
            _full((H, D)), _full((1, D)),
            _full((D, H)), _full((1, H)),
            _full((H, SCORE)), _full((1, SCORE)),
        ],
        out_specs=_full((G, SCORE)),
        out_shape=jax.ShapeDtypeStruct((G, SCORE), jnp.float32),
        scratch_shapes=[pltpu.VMEM((G, H), jnp.float32),
                        pltpu.VMEM((G, 1), jnp.float32)],
        compiler_params=pltpu.CompilerParams(
            dimension_semantics=("arbitrary",)),
    )(x, batch2d, phi_W1, phi_b1.reshape(1, H), phi_W2, phi_b2.reshape(1, D),
      f_W1, f_b1.reshape(1, H), f_W2, f_b2.reshape(1, SCORE))


def kernel(x, edge_index, batch, phi_W1, phi_b1, phi_W2, phi_b2,
           f_W1, f_b1, f_W2, f_b2):
    del edge_index
    return _run(x.astype(jnp.bfloat16), batch.reshape(NB, 1, BN),
                phi_W1.astype(jnp.bfloat16), phi_b1, phi_W2, phi_b2,
                f_W1, f_b1, f_W2, f_b2)


# speedup vs baseline: 1.3821x; 1.3821x over previous
import jax
import jax.numpy as jnp
from jax.experimental import pallas as pl
from jax.experimental.pallas import tpu as pltpu

N = 10000
D = 128
H = 128
G = 64
SCORE = 10

BN = 2000
NB = N // BN


def _pfn_kernel(x_ref, batch_ref, pw1_ref, pb1_ref, pw2_ref, pb2_ref,
                fw1_ref, fb1_ref, fw2_ref, fb2_ref, out_ref,
                p_ref, cnt_ref):
    i = pl.program_id(0)

    @pl.when(i == 0)
    def _init():
        p_ref[...] = jnp.zeros_like(p_ref)
        cnt_ref[...] = jnp.zeros_like(cnt_ref)

    h1 = jax.lax.dot_general(x_ref[...].astype(jnp.bfloat16),
                             pw1_ref[...].astype(jnp.bfloat16),
                             (((1,), (0,)), ((), ())),
                             preferred_element_type=jnp.float32)
    h1 = jnp.maximum(h1 + pb1_ref[...], 0.0)
    onehot_t = (batch_ref[0] ==
                jax.lax.broadcasted_iota(jnp.int32, (G, 1), 0)).astype(jnp.bfloat16)
    p_ref[...] += jax.lax.dot_general(onehot_t, h1.astype(jnp.bfloat16),
                                      (((1,), (0,)), ((), ())),
                                      preferred_element_type=jnp.float32)
    cnt_ref[...] += jnp.sum(onehot_t.astype(jnp.float32), axis=1, keepdims=True)

    @pl.when(i == NB - 1)
    def _tail():
        hp = jax.lax.Precision.HIGHEST
        pooled = jax.lax.dot_general(p_ref[...], pw2_ref[...],
                                     (((1,), (0,)), ((), ())),
                                     preferred_element_type=jnp.float32,
                                     precision=hp)
        pooled = pooled + cnt_ref[...] * pb2_ref[...]
        z = jax.lax.dot_general(pooled, fw1_ref[...], (((1,), (0,)), ((), ())),
                                preferred_element_type=jnp.float32,
                                precision=hp)
        z = jnp.maximum(z + fb1_ref[...], 0.0)
        out_ref[...] = jax.lax.dot_general(z, fw2_ref[...],
                                           (((1,), (0,)), ((), ())),
                                           preferred_element_type=jnp.float32,
                                           precision=hp) + fb2_ref[...]


def _full(shape):
    return pl.BlockSpec(shape, lambda i: (0, 0))


@jax.jit
def _run(x, batch2d, phi_W1, phi_b1, phi_W2, phi_b2, f_W1, f_b1, f_W2, f_b2):
    return pl.pallas_call(
        _pfn_kernel,
        grid=(NB,),
        in_specs=[
            pl.BlockSpec((BN, D), lambda i: (i, 0)),
            pl.BlockSpec((1, 1, BN), lambda i: (i, 0, 0)),
            _full((D, H)), _full((1, H)),
            _full((H, D)), _full((1, D)),
            _full((D, H)), _full((1, H)),
            _full((H, SCORE)), _full((1, SCORE)),
        ],
        out_specs=_full((G, SCORE)),
        out_shape=jax.ShapeDtypeStruct((G, SCORE), jnp.float32),
        scratch_shapes=[pltpu.VMEM((G, H), jnp.float32),
                        pltpu.VMEM((G, 1), jnp.float32)],
        compiler_params=pltpu.CompilerParams(
            dimension_semantics=("arbitrary",)),
    )(x, batch2d, phi_W1, phi_b1.reshape(1, H), phi_W2, phi_b2.reshape(1, D),
      f_W1, f_b1.reshape(1, H), f_W2, f_b2.reshape(1, SCORE))


def kernel(x, edge_index, batch, phi_W1, phi_b1, phi_W2, phi_b2,
           f_W1, f_b1, f_W2, f_b2):
    del edge_index
    return _run(x, batch.reshape(NB, 1, BN), phi_W1, phi_b1, phi_W2, phi_b2,
                f_W1, f_b1, f_W2, f_b2)


# speedup vs baseline: 1.5760x; 1.1403x over previous
import jax
import jax.numpy as jnp
from jax.experimental import pallas as pl
from jax.experimental.pallas import tpu as pltpu

N = 10000
D = 128
H = 128
G = 64
SCORE = 10

BN = 5000
NB = N // BN


def _pfn_kernel(x_ref, batch_ref, pw1_ref, pb1_ref, pw2_ref, pb2_ref,
                fw1_ref, fb1_ref, fw2_ref, fb2_ref, out_ref,
                p_ref, cnt_ref):
    i = pl.program_id(0)

    @pl.when(i == 0)
    def _init():
        p_ref[...] = jnp.zeros_like(p_ref)
        cnt_ref[...] = jnp.zeros_like(cnt_ref)

    h1 = jax.lax.dot_general(x_ref[...], pw1_ref[...], (((1,), (0,)), ((), ())),
                             preferred_element_type=jnp.float32)
    h1 = jnp.maximum(h1 + pb1_ref[...], 0.0)
    onehot_t = (batch_ref[0] ==
                jax.lax.broadcasted_iota(jnp.int32, (G, 1), 0)).astype(jnp.float32)
    p_ref[...] += jax.lax.dot_general(onehot_t, h1, (((1,), (0,)), ((), ())),
                                      preferred_element_type=jnp.float32)
    cnt_ref[...] += jnp.sum(onehot_t, axis=1, keepdims=True)

    @pl.when(i == NB - 1)
    def _tail():
        hp = jax.lax.Precision.HIGHEST
        pooled = jax.lax.dot_general(p_ref[...], pw2_ref[...],
                                     (((1,), (0,)), ((), ())),
                                     preferred_element_type=jnp.float32,
                                     precision=hp)
        pooled = pooled + cnt_ref[...] * pb2_ref[...]
        z = jax.lax.dot_general(pooled, fw1_ref[...], (((1,), (0,)), ((), ())),
                                preferred_element_type=jnp.float32,
                                precision=hp)
        z = jnp.maximum(z + fb1_ref[...], 0.0)
        out_ref[...] = jax.lax.dot_general(z, fw2_ref[...],
                                           (((1,), (0,)), ((), ())),
                                           preferred_element_type=jnp.float32,
                                           precision=hp) + fb2_ref[...]


def _full(shape):
    return pl.BlockSpec(shape, lambda i: (0, 0))


@jax.jit
def _run(x, batch2d, phi_W1, phi_b1, phi_W2, phi_b2, f_W1, f_b1, f_W2, f_b2):
    return pl.pallas_call(
        _pfn_kernel,
        grid=(NB,),
        in_specs=[
            pl.BlockSpec((BN, D), lambda i: (i, 0)),
            pl.BlockSpec((1, 1, BN), lambda i: (i, 0, 0)),
            _full((D, H)), _full((1, H)),
            _full((H, D)), _full((1, D)),
            _full((D, H)), _full((1, H)),
            _full((H, SCORE)), _full((1, SCORE)),
        ],
        out_specs=_full((G, SCORE)),
        out_shape=jax.ShapeDtypeStruct((G, SCORE), jnp.float32),
        scratch_shapes=[pltpu.VMEM((G, H), jnp.float32),
                        pltpu.VMEM((G, 1), jnp.float32)],
        compiler_params=pltpu.CompilerParams(
            dimension_semantics=("arbitrary",)),
    )(x, batch2d, phi_W1, phi_b1.reshape(1, H), phi_W2, phi_b2.reshape(1, D),
      f_W1, f_b1.reshape(1, H), f_W2, f_b2.reshape(1, SCORE))


def kernel(x, edge_index, batch, phi_W1, phi_b1, phi_W2, phi_b2,
           f_W1, f_b1, f_W2, f_b2):
    del edge_index
    return _run(x, batch.reshape(NB, 1, BN), phi_W1, phi_b1, phi_W2, phi_b2,
                f_W1, f_b1, f_W2, f_b2)


# speedup vs baseline: 1.6183x; 1.0269x over previous
import jax
import jax.numpy as jnp
from jax.experimental import pallas as pl
from jax.experimental.pallas import tpu as pltpu

N = 10000
D = 128
H = 128
G = 64
SCORE = 10

BN = 10000
NB = N // BN


def _pfn_kernel(x_ref, batch_ref, pw1_ref, pb1_ref, pw2_ref, pb2_ref,
                fw1_ref, fb1_ref, fw2_ref, fb2_ref, out_ref,
                p_ref, cnt_ref):
    i = pl.program_id(0)

    @pl.when(i == 0)
    def _init():
        p_ref[...] = jnp.zeros_like(p_ref)
        cnt_ref[...] = jnp.zeros_like(cnt_ref)

    h1 = jax.lax.dot_general(x_ref[...], pw1_ref[...], (((1,), (0,)), ((), ())),
                             preferred_element_type=jnp.float32)
    h1 = jnp.maximum(h1 + pb1_ref[...], 0.0)
    onehot_t = (batch_ref[0] ==
                jax.lax.broadcasted_iota(jnp.int32, (G, 1), 0)).astype(jnp.float32)
    p_ref[...] += jax.lax.dot_general(onehot_t, h1, (((1,), (0,)), ((), ())),
                                      preferred_element_type=jnp.float32)
    cnt_ref[...] += jnp.sum(onehot_t, axis=1, keepdims=True)

    @pl.when(i == NB - 1)
    def _tail():
        hp = jax.lax.Precision.HIGHEST
        pooled = jax.lax.dot_general(p_ref[...], pw2_ref[...],
                                     (((1,), (0,)), ((), ())),
                                     preferred_element_type=jnp.float32,
                                     precision=hp)
        pooled = pooled + cnt_ref[...] * pb2_ref[...]
        z = jax.lax.dot_general(pooled, fw1_ref[...], (((1,), (0,)), ((), ())),
                                preferred_element_type=jnp.float32,
                                precision=hp)
        z = jnp.maximum(z + fb1_ref[...], 0.0)
        out_ref[...] = jax.lax.dot_general(z, fw2_ref[...],
                                           (((1,), (0,)), ((), ())),
                                           preferred_element_type=jnp.float32,
                                           precision=hp) + fb2_ref[...]


def _full(shape):
    return pl.BlockSpec(shape, lambda i: (0, 0))


@jax.jit
def _run(x, batch2d, phi_W1, phi_b1, phi_W2, phi_b2, f_W1, f_b1, f_W2, f_b2):
    return pl.pallas_call(
        _pfn_kernel,
        grid=(NB,),
        in_specs=[
            pl.BlockSpec((BN, D), lambda i: (i, 0)),
            pl.BlockSpec((1, 1, BN), lambda i: (i, 0, 0)),
            _full((D, H)), _full((1, H)),
            _full((H, D)), _full((1, D)),
            _full((D, H)), _full((1, H)),
            _full((H, SCORE)), _full((1, SCORE)),
        ],
        out_specs=_full((G, SCORE)),
        out_shape=jax.ShapeDtypeStruct((G, SCORE), jnp.float32),
        scratch_shapes=[pltpu.VMEM((G, H), jnp.float32),
                        pltpu.VMEM((G, 1), jnp.float32)],
        compiler_params=pltpu.CompilerParams(
            dimension_semantics=("arbitrary",)),
    )(x, batch2d, phi_W1, phi_b1.reshape(1, H), phi_W2, phi_b2.reshape(1, D),
      f_W1, f_b1.reshape(1, H), f_W2, f_b2.reshape(1, SCORE))


def kernel(x, edge_index, batch, phi_W1, phi_b1, phi_W2, phi_b2,
           f_W1, f_b1, f_W2, f_b2):
    del edge_index
    return _run(x, batch.reshape(NB, 1, BN), phi_W1, phi_b1, phi_W2, phi_b2,
                f_W1, f_b1, f_W2, f_b2)


# speedup vs baseline: 1.6190x; 1.0004x over previous
import jax
import jax.numpy as jnp
from jax.experimental import pallas as pl
from jax.experimental.pallas import tpu as pltpu

N = 10000
D = 128
H = 128
G = 64
SCORE = 10


def _pfn_kernel(x_ref, batch_ref, pw1_ref, pb1_ref, pw2_ref, pb2_ref,
                fw1_ref, fb1_ref, fw2_ref, fb2_ref, out_ref):
    h1 = jax.lax.dot_general(x_ref[...], pw1_ref[...], (((1,), (0,)), ((), ())),
                             preferred_element_type=jnp.float32)
    h1 = jnp.maximum(h1 + pb1_ref[...], 0.0)
    onehot_t = (batch_ref[...] ==
                jax.lax.broadcasted_iota(jnp.int32, (G, 1), 0)).astype(jnp.float32)
    p = jax.lax.dot_general(onehot_t, h1, (((1,), (0,)), ((), ())),
                            preferred_element_type=jnp.float32)
    cnt = jnp.sum(onehot_t, axis=1, keepdims=True)
    hp = jax.lax.Precision.HIGHEST
    pooled = jax.lax.dot_general(p, pw2_ref[...], (((1,), (0,)), ((), ())),
                                 preferred_element_type=jnp.float32,
                                 precision=hp)
    pooled = pooled + cnt * pb2_ref[...]
    z = jax.lax.dot_general(pooled, fw1_ref[...], (((1,), (0,)), ((), ())),
                            preferred_element_type=jnp.float32, precision=hp)
    z = jnp.maximum(z + fb1_ref[...], 0.0)
    out_ref[...] = jax.lax.dot_general(z, fw2_ref[...], (((1,), (0,)), ((), ())),
                                       preferred_element_type=jnp.float32,
                                       precision=hp) + fb2_ref[...]


@jax.jit
def _run(x, batch2d, phi_W1, phi_b1, phi_W2, phi_b2, f_W1, f_b1, f_W2, f_b2):
    return pl.pallas_call(
        _pfn_kernel,
        out_shape=jax.ShapeDtypeStruct((G, SCORE), jnp.float32),
    )(x, batch2d, phi_W1, phi_b1.reshape(1, H), phi_W2, phi_b2.reshape(1, D),
      f_W1, f_b1.reshape(1, H), f_W2, f_b2.reshape(1, SCORE))


def kernel(x, edge_index, batch, phi_W1, phi_b1, phi_W2, phi_b2,
           f_W1, f_b1, f_W2, f_b2):
    del edge_index
    return _run(x, batch.reshape(1, N), phi_W1, phi_b1, phi_W2, phi_b2,
                f_W1, f_b1, f_W2, f_b2)
